# baseline (speedup 1.0000x reference)
import jax
import jax.numpy as jnp
from jax import lax
from jax.experimental import pallas as pl
from jax.experimental.pallas import tpu as pltpu
from jax.experimental.pallas import tpu_sc as plsc

D = 64
H = 50
B = 4096
NW = 32
BPW = B // NW
HS = 56
EG = 2
NG = BPW // EG
GLOBAL_MEAN = 3.5


def _iota16():
    return lax.iota(jnp.int32, 16)


def _full16(x):
    return jnp.full((16,), x, dtype=jnp.int32)


def _body(user_h, item_h, midx_h, len_h, uemb_h, iemb_h, impl_h, ub_h, ib_h,
          lut_h, out_h,
          u_v, i_v, gidx_v, len_v, ue_v, ie_v, ub_v, ib_v, lut_v,
          buf0, buf1, out_v, sem0, sb0, sb1):
    bufs = (buf0, buf1)
    bsem = (sb0, sb1)
    wid = lax.axis_index("s") * 2 + lax.axis_index("c")
    base = wid * BPW

    pltpu.sync_copy(user_h.at[pl.ds(base, BPW)], u_v)
    pltpu.sync_copy(item_h.at[pl.ds(base, BPW)], i_v)
    pltpu.sync_copy(lut_h, lut_v)
    pltpu.sync_copy(midx_h.at[pl.ds(wid * NG, NG), :], gidx_v)

    cps = [
        pltpu.async_copy(len_h.at[u_v], len_v, sem0),
        pltpu.async_copy(uemb_h.at[u_v], ue_v, sem0),
        pltpu.async_copy(iemb_h.at[i_v], ie_v, sem0),
        pltpu.async_copy(ub_h.at[u_v], ub_v, sem0),
        pltpu.async_copy(ib_h.at[i_v], ib_v, sem0),
    ]
    for cp in cps:
        cp.wait()

    def fire(g, b):
        pltpu.async_copy(impl_h.at[gidx_v.at[g]], bufs[b], bsem[b])

    def wait(g, b):
        pltpu.make_async_copy(impl_h.at[gidx_v.at[g]], bufs[b],
                              bsem[b]).wait()

    def compute(g, buf):
        for j in range(EG):
            e = g * EG + j
            e16 = _full16(e)
            len16 = plsc.load_gather(len_v, [e16])
            rs = plsc.load_gather(lut_v, [len16])
            coef = jnp.float32(HS) - len16.astype(jnp.float32)
            r0 = j * HS
            p = None
            for c in range(4):
                col = pl.ds(c * 16, 16)
                a = [buf[r0 + k, col] for k in range(4)]
                for s in range(4, HS, 4):
                    for k in range(4):
                        a[k] = a[k] + buf[r0 + s + k, col]
                acc = (a[0] + a[1]) + (a[2] + a[3]) - coef * buf[r0, col]
                c16 = _iota16() + c * 16
                tu = plsc.load_gather(ue_v, [e16, c16]) + acc * rs
                t = tu * plsc.load_gather(ie_v, [e16, c16])
                p = t if p is None else p + t
            tot = plsc.cumsum(p)
            ubx = plsc.load_gather(ub_v, [e16])
            ibx = plsc.load_gather(ib_v, [e16])
            rating = tot + (jnp.float32(GLOBAL_MEAN) + ubx + ibx)
            plsc.store_scatter(out_v, [e16], rating, mask=_iota16() == 15)

    for b in range(2):
        fire(b, b)

    @pl.loop(0, NG - 2, step=2)
    def ring(g):
        for b in range(2):
            gg = g + b
            wait(gg, b)
            compute(gg, bufs[b])
            fire(gg + 2, b)

    for b in range(2):
        wait(NG - 2 + b, b)
        compute(NG - 2 + b, bufs[b])

    pltpu.sync_copy(out_v, out_h.at[pl.ds(base, BPW)])


@jax.jit
def kernel(inputs, rated_items, lengths, user_emb, item_emb, impl_emb,
           user_bias, item_bias):
    user = inputs[:, 0]
    item = inputs[:, 1]
    lut = lax.rsqrt(jnp.maximum(jnp.arange(64, dtype=jnp.float32), 1.0))

    ri = jnp.take(rated_items, user, axis=0)
    len_u = jnp.take(lengths, user, axis=0)
    ri56 = jnp.concatenate(
        [ri, jnp.broadcast_to(ri[:, :1], (B, HS - H))], axis=1)
    h56 = jnp.arange(HS, dtype=len_u.dtype)[None, :]
    midx = jnp.where(h56 < len_u[:, None], ri56, ri[:, :1])
    midx = midx.reshape(B // EG, EG * HS)

    mesh = plsc.VectorSubcoreMesh(core_axis_name="c", subcore_axis_name="s",
                                  num_cores=2, num_subcores=16)
    fn = pl.kernel(
        _body,
        out_type=jax.ShapeDtypeStruct((B,), jnp.float32),
        mesh=mesh,
        compiler_params=pltpu.CompilerParams(needs_layout_passes=False,
                                             use_tc_tiling_on_sc=False),
        scratch_types=[
            pltpu.VMEM((BPW,), jnp.int32),
            pltpu.VMEM((BPW,), jnp.int32),
            pltpu.VMEM((NG, EG * HS), jnp.int32),
            pltpu.VMEM((BPW,), jnp.int32),
            pltpu.VMEM((BPW, D), jnp.float32),
            pltpu.VMEM((BPW, D), jnp.float32),
            pltpu.VMEM((BPW,), jnp.float32),
            pltpu.VMEM((BPW,), jnp.float32),
            pltpu.VMEM((64,), jnp.float32),
            pltpu.VMEM((EG * HS, D), jnp.float32),
            pltpu.VMEM((EG * HS, D), jnp.float32),
            pltpu.VMEM((BPW,), jnp.float32),
            pltpu.SemaphoreType.DMA,
            pltpu.SemaphoreType.DMA,
            pltpu.SemaphoreType.DMA,
        ],
    )
    out = fn(user, item, midx, lengths, user_emb, item_emb, impl_emb,
             user_bias[:, 0], item_bias[:, 0], lut)
    return out.reshape(B, 1)


# speedup vs baseline: 1.0063x; 1.0063x over previous
import jax
import jax.numpy as jnp
from jax import lax
from jax.experimental import pallas as pl
from jax.experimental.pallas import tpu as pltpu
from jax.experimental.pallas import tpu_sc as plsc

D = 64
H = 50
B = 4096
NW = 32
BPW = B // NW
HS = 56
EG = 2
NG = BPW // EG
GLOBAL_MEAN = 3.5


def _iota16():
    return lax.iota(jnp.int32, 16)


def _full16(x):
    return jnp.full((16,), x, dtype=jnp.int32)


def _body(user_h, item_h, midx_h, len_h, uemb_h, iemb_h, impl_h, ub_h, ib_h,
          lut_h, out_h,
          u_v, i_v, gidx_v, len_v, ue_v, ie_v, ub_v, ib_v, lut_v,
          buf0, buf1, out_v, sem0, sb0, sb1):
    bufs = (buf0, buf1)
    bsem = (sb0, sb1)
    wid = lax.axis_index("s") * 2 + lax.axis_index("c")
    base = wid * BPW

    pltpu.sync_copy(user_h.at[pl.ds(base, BPW)], u_v)
    pltpu.sync_copy(item_h.at[pl.ds(base, BPW)], i_v)
    pltpu.sync_copy(lut_h, lut_v)
    pltpu.sync_copy(midx_h.at[pl.ds(wid * NG, NG), :], gidx_v)

    cps = [
        pltpu.async_copy(len_h.at[u_v], len_v, sem0),
        pltpu.async_copy(uemb_h.at[u_v], ue_v, sem0),
        pltpu.async_copy(iemb_h.at[i_v], ie_v, sem0),
        pltpu.async_copy(ub_h.at[u_v], ub_v, sem0),
        pltpu.async_copy(ib_h.at[i_v], ib_v, sem0),
    ]
    for cp in cps:
        cp.wait()

    def fire(g, b):
        pltpu.async_copy(impl_h.at[gidx_v.at[g]], bufs[b], bsem[b])

    def wait(g, b):
        pltpu.make_async_copy(impl_h.at[gidx_v.at[g]], bufs[b],
                              bsem[b]).wait()

    def compute(g, buf):
        for j in range(EG):
            e = g * EG + j
            e16 = _full16(e)
            len16 = plsc.load_gather(len_v, [e16])
            rs = plsc.load_gather(lut_v, [len16])
            coef = jnp.float32(HS) - len16.astype(jnp.float32)
            r0 = j * HS
            p = None
            for c in range(4):
                col = pl.ds(c * 16, 16)
                a0 = buf[r0, col]
                a1 = buf[r0 + 1, col]
                for s in range(2, HS, 2):
                    a0 = a0 + buf[r0 + s, col]
                    a1 = a1 + buf[r0 + s + 1, col]
                acc = a0 + a1 - coef * buf[r0, col]
                c16 = _iota16() + c * 16
                tu = plsc.load_gather(ue_v, [e16, c16]) + acc * rs
                t = tu * plsc.load_gather(ie_v, [e16, c16])
                p = t if p is None else p + t
            tot = plsc.cumsum(p)
            ubx = plsc.load_gather(ub_v, [e16])
            ibx = plsc.load_gather(ib_v, [e16])
            rating = tot + (jnp.float32(GLOBAL_MEAN) + ubx + ibx)
            plsc.store_scatter(out_v, [e16], rating, mask=_iota16() == 15)

    for b in range(2):
        fire(b, b)

    @pl.loop(0, NG - 2, step=2)
    def ring(g):
        for b in range(2):
            gg = g + b
            wait(gg, b)
            compute(gg, bufs[b])
            fire(gg + 2, b)

    for b in range(2):
        wait(NG - 2 + b, b)
        compute(NG - 2 + b, bufs[b])

    pltpu.sync_copy(out_v, out_h.at[pl.ds(base, BPW)])


@jax.jit
def kernel(inputs, rated_items, lengths, user_emb, item_emb, impl_emb,
           user_bias, item_bias):
    user = inputs[:, 0]
    item = inputs[:, 1]
    lut = lax.rsqrt(jnp.maximum(jnp.arange(64, dtype=jnp.float32), 1.0))

    ri = jnp.take(rated_items, user, axis=0)
    len_u = jnp.take(lengths, user, axis=0)
    ri56 = jnp.concatenate(
        [ri, jnp.broadcast_to(ri[:, :1], (B, HS - H))], axis=1)
    h56 = jnp.arange(HS, dtype=len_u.dtype)[None, :]
    midx = jnp.where(h56 < len_u[:, None], ri56, ri[:, :1])
    midx = midx.reshape(B // EG, EG * HS)

    mesh = plsc.VectorSubcoreMesh(core_axis_name="c", subcore_axis_name="s",
                                  num_cores=2, num_subcores=16)
    fn = pl.kernel(
        _body,
        out_type=jax.ShapeDtypeStruct((B,), jnp.float32),
        mesh=mesh,
        compiler_params=pltpu.CompilerParams(needs_layout_passes=False,
                                             use_tc_tiling_on_sc=False),
        scratch_types=[
            pltpu.VMEM((BPW,), jnp.int32),
            pltpu.VMEM((BPW,), jnp.int32),
            pltpu.VMEM((NG, EG * HS), jnp.int32),
            pltpu.VMEM((BPW,), jnp.int32),
            pltpu.VMEM((BPW, D), jnp.float32),
            pltpu.VMEM((BPW, D), jnp.float32),
            pltpu.VMEM((BPW,), jnp.float32),
            pltpu.VMEM((BPW,), jnp.float32),
            pltpu.VMEM((64,), jnp.float32),
            pltpu.VMEM((EG * HS, D), jnp.float32),
            pltpu.VMEM((EG * HS, D), jnp.float32),
            pltpu.VMEM((BPW,), jnp.float32),
            pltpu.SemaphoreType.DMA,
            pltpu.SemaphoreType.DMA,
            pltpu.SemaphoreType.DMA,
        ],
    )
    out = fn(user, item, midx, lengths, user_emb, item_emb, impl_emb,
             user_bias[:, 0], item_bias[:, 0], lut)
    return out.reshape(B, 1)
